# baseline (speedup 1.0000x reference)
import dataclasses
import functools

import jax
import jax.numpy as jnp
from jax import lax
from jax.experimental import pallas as pl
from jax.experimental.pallas import tpu as pltpu
from jax.experimental.pallas import tpu_sc as plsc

NUM_CLASSES = 119
PADW = 128
LANES = 16
NC, NS = 2, 16
NW = NC * NS
CHUNK = 256
NBUF = 2
NPART = 4
RB = 1024


def _onehot_sc_dense(idx_flat: jax.Array) -> jax.Array:
    total = idx_flat.shape[0]
    rows_per_w = total // NW
    n_chunks = rows_per_w // CHUNK
    assert rows_per_w % CHUNK == 0 and n_chunks % NBUF == 0
    assert CHUNK % LANES == 0 and rows_per_w % 8 == 0

    zeros_tile = jnp.zeros((CHUNK, PADW), jnp.float32)
    mesh = plsc.VectorSubcoreMesh(core_axis_name="c", subcore_axis_name="s")
    cp = pltpu.CompilerParams()
    if "needs_layout_passes" in pltpu.CompilerParams.__dataclass_fields__:
        cp = dataclasses.replace(cp, needs_layout_passes=False)

    @functools.partial(
        pl.kernel,
        mesh=mesh,
        compiler_params=cp,
        out_type=jax.ShapeDtypeStruct((total, PADW), jnp.float32),
        scratch_types=[
            pltpu.VMEM((rows_per_w,), jnp.int32),
            pltpu.VMEM((NBUF, CHUNK, PADW), jnp.float32),
        ] + [pltpu.SemaphoreType.DMA] * NBUF,
    )
    def kern(idx_hbm, zeros_hbm, out_hbm, idx_v, bufs, *sems):
        wid = lax.axis_index("s") * NC + lax.axis_index("c")
        base = wid * rows_per_w

        for b in range(NBUF):
            pltpu.make_async_copy(zeros_hbm, bufs.at[b], sems[b]).start()
        pltpu.sync_copy(idx_hbm.at[pl.ds(wid * rows_per_w, rows_per_w)],
                        idx_v)
        for b in range(NBUF):
            pltpu.make_async_copy(zeros_hbm, bufs.at[b], sems[b]).wait()

        ones_v = jnp.ones((LANES,), jnp.float32)
        zeros_v = jnp.zeros((LANES,), jnp.float32)
        row_vecs = [lax.iota(jnp.int32, LANES) + g * LANES
                    for g in range(CHUNK // LANES)]

        def scatter_chunk(g, b, vals):
            for sub in range(CHUNK // LANES):
                cols = idx_v[pl.ds(g * CHUNK + sub * LANES, LANES)]
                plsc.store_scatter(bufs.at[b], [row_vecs[sub], cols], vals)

        def out_copy(g, b):
            return pltpu.make_async_copy(
                bufs.at[b],
                out_hbm.at[pl.ds(base + g * CHUNK, CHUNK)],
                sems[b])

        @pl.loop(0, n_chunks, step=NBUF)
        def _(g0):
            for b in range(NBUF):
                g = g0 + b

                @pl.when(g >= NBUF)
                def _():
                    out_copy(g - NBUF, b).wait()
                    scatter_chunk(g - NBUF, b, zeros_v)

                scatter_chunk(g, b, ones_v)
                out_copy(g, b).start()

        for b in range(NBUF):
            out_copy(n_chunks - NBUF + b, b).wait()

    return kern(idx_flat, zeros_tile)


def _compact_tc(dense_p, prev, total, blk0, nblk):
    tc_params = pltpu.CompilerParams(dimension_semantics=("parallel",))

    if prev is None:
        def body(d_ref, o_ref):
            o_ref[...] = d_ref[:, :NUM_CLASSES]
        in_specs = [pl.BlockSpec((RB, PADW), lambda i: (i, 0))]
        args = (dense_p,)
        aliases = {}
    else:
        def body(d_ref, _prev_ref, o_ref):
            o_ref[...] = d_ref[:, :NUM_CLASSES]
        in_specs = [pl.BlockSpec((RB, PADW), lambda i: (i, 0)),
                    pl.BlockSpec(memory_space=pl.ANY)]
        args = (dense_p, prev)
        aliases = {1: 0}

    return pl.pallas_call(
        body,
        grid=(nblk,),
        in_specs=in_specs,
        out_specs=pl.BlockSpec((RB, NUM_CLASSES),
                               lambda i, blk0=blk0: (blk0 + i, 0)),
        out_shape=jax.ShapeDtypeStruct((total, NUM_CLASSES), jnp.float32),
        input_output_aliases=aliases,
        compiler_params=tc_params,
    )(*args)


def kernel(atomic_numbers):
    idx_flat = atomic_numbers.astype(jnp.int32).reshape(-1)
    n_rows, row_w = atomic_numbers.shape
    total = idx_flat.shape[0]
    part = total // NPART
    assert part % (NW * CHUNK) == 0 and part % RB == 0

    out = None
    for p in range(NPART):
        dense_p = _onehot_sc_dense(idx_flat[p * part:(p + 1) * part])
        out = _compact_tc(dense_p, out, total,
                          blk0=p * part // RB, nblk=part // RB)
    return out.reshape(n_rows, row_w, NUM_CLASSES)


# speedup vs baseline: 1.0151x; 1.0151x over previous
import dataclasses
import functools

import jax
import jax.numpy as jnp
from jax import lax
from jax.experimental import pallas as pl
from jax.experimental.pallas import tpu as pltpu
from jax.experimental.pallas import tpu_sc as plsc

NUM_CLASSES = 119
PADW = 128
LANES = 16
NC, NS = 2, 16
NW = NC * NS
CHUNK = 256
NBUF = 2
NPART = 1
RB = 1024


def _onehot_sc_dense(idx_flat: jax.Array) -> jax.Array:
    total = idx_flat.shape[0]
    rows_per_w = total // NW
    n_chunks = rows_per_w // CHUNK
    assert rows_per_w % CHUNK == 0 and n_chunks % NBUF == 0
    assert CHUNK % LANES == 0 and rows_per_w % 8 == 0

    zeros_tile = jnp.zeros((CHUNK, PADW), jnp.float32)
    mesh = plsc.VectorSubcoreMesh(core_axis_name="c", subcore_axis_name="s")
    cp = pltpu.CompilerParams()
    if "needs_layout_passes" in pltpu.CompilerParams.__dataclass_fields__:
        cp = dataclasses.replace(cp, needs_layout_passes=False)

    @functools.partial(
        pl.kernel,
        mesh=mesh,
        compiler_params=cp,
        out_type=jax.ShapeDtypeStruct((total, PADW), jnp.float32),
        scratch_types=[
            pltpu.VMEM((rows_per_w,), jnp.int32),
            pltpu.VMEM((NBUF, CHUNK, PADW), jnp.float32),
        ] + [pltpu.SemaphoreType.DMA] * NBUF,
    )
    def kern(idx_hbm, zeros_hbm, out_hbm, idx_v, bufs, *sems):
        wid = lax.axis_index("s") * NC + lax.axis_index("c")
        base = wid * rows_per_w

        for b in range(NBUF):
            pltpu.make_async_copy(zeros_hbm, bufs.at[b], sems[b]).start()
        pltpu.sync_copy(idx_hbm.at[pl.ds(wid * rows_per_w, rows_per_w)],
                        idx_v)
        for b in range(NBUF):
            pltpu.make_async_copy(zeros_hbm, bufs.at[b], sems[b]).wait()

        ones_v = jnp.ones((LANES,), jnp.float32)
        zeros_v = jnp.zeros((LANES,), jnp.float32)
        row_vecs = [lax.iota(jnp.int32, LANES) + g * LANES
                    for g in range(CHUNK // LANES)]

        def scatter_chunk(g, b, vals):
            for sub in range(CHUNK // LANES):
                cols = idx_v[pl.ds(g * CHUNK + sub * LANES, LANES)]
                plsc.store_scatter(bufs.at[b], [row_vecs[sub], cols], vals)

        def out_copy(g, b):
            return pltpu.make_async_copy(
                bufs.at[b],
                out_hbm.at[pl.ds(base + g * CHUNK, CHUNK)],
                sems[b])

        @pl.loop(0, n_chunks, step=NBUF)
        def _(g0):
            for b in range(NBUF):
                g = g0 + b

                @pl.when(g >= NBUF)
                def _():
                    out_copy(g - NBUF, b).wait()
                    scatter_chunk(g - NBUF, b, zeros_v)

                scatter_chunk(g, b, ones_v)
                out_copy(g, b).start()

        for b in range(NBUF):
            out_copy(n_chunks - NBUF + b, b).wait()

    return kern(idx_flat, zeros_tile)


def _compact_tc(dense_p, prev, total, blk0, nblk):
    tc_params = pltpu.CompilerParams(dimension_semantics=("parallel",))

    if prev is None:
        def body(d_ref, o_ref):
            o_ref[...] = d_ref[:, :NUM_CLASSES]
        in_specs = [pl.BlockSpec((RB, PADW), lambda i: (i, 0))]
        args = (dense_p,)
        aliases = {}
    else:
        def body(d_ref, _prev_ref, o_ref):
            o_ref[...] = d_ref[:, :NUM_CLASSES]
        in_specs = [pl.BlockSpec((RB, PADW), lambda i: (i, 0)),
                    pl.BlockSpec(memory_space=pl.ANY)]
        args = (dense_p, prev)
        aliases = {1: 0}

    return pl.pallas_call(
        body,
        grid=(nblk,),
        in_specs=in_specs,
        out_specs=pl.BlockSpec((RB, NUM_CLASSES),
                               lambda i, blk0=blk0: (blk0 + i, 0)),
        out_shape=jax.ShapeDtypeStruct((total, NUM_CLASSES), jnp.float32),
        input_output_aliases=aliases,
        compiler_params=tc_params,
    )(*args)


def kernel(atomic_numbers):
    idx_flat = atomic_numbers.astype(jnp.int32).reshape(-1)
    n_rows, row_w = atomic_numbers.shape
    total = idx_flat.shape[0]
    part = total // NPART
    assert part % (NW * CHUNK) == 0 and part % RB == 0

    out = None
    for p in range(NPART):
        dense_p = _onehot_sc_dense(idx_flat[p * part:(p + 1) * part])
        out = _compact_tc(dense_p, out, total,
                          blk0=p * part // RB, nblk=part // RB)
    return out.reshape(n_rows, row_w, NUM_CLASSES)


# speedup vs baseline: 2.2619x; 2.2282x over previous
import dataclasses
import functools

import jax
import jax.numpy as jnp
from jax import lax
from jax.experimental import pallas as pl
from jax.experimental.pallas import tpu as pltpu
from jax.experimental.pallas import tpu_sc as plsc

NUM_CLASSES = 119
PADW = 128
LANES = 16
NC, NS = 2, 16
NW = NC * NS
CHUNK = 256
NBUF = 2
NPART = 1
RB = 1024


def _onehot_sc_dense(idx_flat: jax.Array) -> jax.Array:
    total = idx_flat.shape[0]
    rows_per_w = total // NW
    n_chunks = rows_per_w // CHUNK
    assert rows_per_w % CHUNK == 0 and n_chunks % NBUF == 0
    assert CHUNK % LANES == 0 and rows_per_w % 8 == 0

    zeros_tile = jnp.zeros((CHUNK, PADW), jnp.float32)
    mesh = plsc.VectorSubcoreMesh(core_axis_name="c", subcore_axis_name="s")
    cp = pltpu.CompilerParams()
    if "needs_layout_passes" in pltpu.CompilerParams.__dataclass_fields__:
        cp = dataclasses.replace(cp, needs_layout_passes=False)

    @functools.partial(
        pl.kernel,
        mesh=mesh,
        compiler_params=cp,
        out_type=jax.ShapeDtypeStruct((total, PADW), jnp.float32),
        scratch_types=[
            pltpu.VMEM((rows_per_w,), jnp.int32),
            pltpu.VMEM((NBUF, CHUNK, PADW), jnp.float32),
        ] + [pltpu.SemaphoreType.DMA] * NBUF,
    )
    def kern(idx_hbm, zeros_hbm, out_hbm, idx_v, bufs, *sems):
        wid = lax.axis_index("s") * NC + lax.axis_index("c")
        base = wid * rows_per_w

        for b in range(NBUF):
            pltpu.make_async_copy(zeros_hbm, bufs.at[b], sems[b]).start()
        pltpu.sync_copy(idx_hbm.at[pl.ds(wid * rows_per_w, rows_per_w)],
                        idx_v)
        for b in range(NBUF):
            pltpu.make_async_copy(zeros_hbm, bufs.at[b], sems[b]).wait()

        ones_v = jnp.ones((LANES,), jnp.float32)
        zeros_v = jnp.zeros((LANES,), jnp.float32)
        row_vecs = [lax.iota(jnp.int32, LANES) + g * LANES
                    for g in range(CHUNK // LANES)]

        def scatter_chunk(g, b, vals):
            for sub in range(CHUNK // LANES):
                cols = idx_v[pl.ds(g * CHUNK + sub * LANES, LANES)]
                plsc.store_scatter(bufs.at[b], [row_vecs[sub], cols], vals)

        def out_copy(g, b):
            return pltpu.make_async_copy(
                bufs.at[b],
                out_hbm.at[pl.ds(base + g * CHUNK, CHUNK)],
                sems[b])

        @pl.loop(0, n_chunks, step=NBUF)
        def _(g0):
            for b in range(NBUF):
                g = g0 + b

                @pl.when(g >= NBUF)
                def _():
                    out_copy(g - NBUF, b).wait()
                    scatter_chunk(g - NBUF, b, zeros_v)

                scatter_chunk(g, b, ones_v)
                out_copy(g, b).start()

        for b in range(NBUF):
            out_copy(n_chunks - NBUF + b, b).wait()

    return kern(idx_flat, zeros_tile)


def _compact_tc(dense_p, prev, total, blk0, nblk):
    tc_params = pltpu.CompilerParams(dimension_semantics=("parallel",))

    if prev is None:
        def body(d_ref, o_ref):
            o_ref[...] = d_ref[:, :NUM_CLASSES]
        in_specs = [pl.BlockSpec((RB, PADW), lambda i: (i, 0))]
        args = (dense_p,)
        aliases = {}
    else:
        def body(d_ref, _prev_ref, o_ref):
            o_ref[...] = d_ref[:, :NUM_CLASSES]
        in_specs = [pl.BlockSpec((RB, PADW), lambda i: (i, 0)),
                    pl.BlockSpec(memory_space=pl.ANY)]
        args = (dense_p, prev)
        aliases = {1: 0}

    return pl.pallas_call(
        body,
        grid=(nblk,),
        in_specs=in_specs,
        out_specs=pl.BlockSpec((RB, NUM_CLASSES),
                               lambda i, blk0=blk0: (blk0 + i, 0)),
        out_shape=jax.ShapeDtypeStruct((total, NUM_CLASSES), jnp.float32),
        input_output_aliases=aliases,
        compiler_params=tc_params,
    )(*args)


def kernel(atomic_numbers):
    idx_flat = atomic_numbers.astype(jnp.int32).reshape(-1)
    n_rows, row_w = atomic_numbers.shape
    total = idx_flat.shape[0]
    part = total // NPART
    assert part % (NW * CHUNK) == 0 and part % RB == 0

    dense = _onehot_sc_dense(idx_flat)
    out = dense[:, :NUM_CLASSES]
    return out.reshape(n_rows, row_w, NUM_CLASSES)


# speedup vs baseline: 2.2657x; 1.0017x over previous
import dataclasses
import functools

import jax
import jax.numpy as jnp
from jax import lax
from jax.experimental import pallas as pl
from jax.experimental.pallas import tpu as pltpu
from jax.experimental.pallas import tpu_sc as plsc

NUM_CLASSES = 119
LANES = 16
NC, NS = 2, 16
NW = NC * NS
CHUNK = 256
NBUF = 2


def _onehot_sc(idx_flat: jax.Array) -> jax.Array:
    total = idx_flat.shape[0]
    rows_per_w = total // NW
    n_chunks = rows_per_w // CHUNK
    assert rows_per_w % CHUNK == 0 and n_chunks % NBUF == 0
    assert CHUNK % LANES == 0 and rows_per_w % 8 == 0

    zeros_tile = jnp.zeros((CHUNK, NUM_CLASSES), jnp.float32)
    mesh = plsc.VectorSubcoreMesh(core_axis_name="c", subcore_axis_name="s")
    cp = pltpu.CompilerParams()
    if "needs_layout_passes" in pltpu.CompilerParams.__dataclass_fields__:
        cp = dataclasses.replace(cp, needs_layout_passes=False)

    @functools.partial(
        pl.kernel,
        mesh=mesh,
        compiler_params=cp,
        out_type=jax.ShapeDtypeStruct((total, NUM_CLASSES), jnp.float32),
        scratch_types=[
            pltpu.VMEM((rows_per_w,), jnp.int32),
            pltpu.VMEM((NBUF, CHUNK, NUM_CLASSES), jnp.float32),
        ] + [pltpu.SemaphoreType.DMA] * NBUF,
    )
    def kern(idx_hbm, zeros_hbm, out_hbm, idx_v, bufs, *sems):
        wid = lax.axis_index("s") * NC + lax.axis_index("c")
        base = wid * rows_per_w

        for b in range(NBUF):
            pltpu.make_async_copy(zeros_hbm, bufs.at[b], sems[b]).start()
        pltpu.sync_copy(idx_hbm.at[pl.ds(wid * rows_per_w, rows_per_w)],
                        idx_v)
        for b in range(NBUF):
            pltpu.make_async_copy(zeros_hbm, bufs.at[b], sems[b]).wait()

        ones_v = jnp.ones((LANES,), jnp.float32)
        zeros_v = jnp.zeros((LANES,), jnp.float32)
        row_vecs = [lax.iota(jnp.int32, LANES) + g * LANES
                    for g in range(CHUNK // LANES)]

        def scatter_chunk(g, b, vals):
            for sub in range(CHUNK // LANES):
                cols = idx_v[pl.ds(g * CHUNK + sub * LANES, LANES)]
                plsc.store_scatter(bufs.at[b], [row_vecs[sub], cols], vals)

        def out_copy(g, b):
            return pltpu.make_async_copy(
                bufs.at[b],
                out_hbm.at[pl.ds(base + g * CHUNK, CHUNK)],
                sems[b])

        @pl.loop(0, n_chunks, step=NBUF)
        def _(g0):
            for b in range(NBUF):
                g = g0 + b

                @pl.when(g >= NBUF)
                def _():
                    out_copy(g - NBUF, b).wait()
                    scatter_chunk(g - NBUF, b, zeros_v)

                scatter_chunk(g, b, ones_v)
                out_copy(g, b).start()

        for b in range(NBUF):
            out_copy(n_chunks - NBUF + b, b).wait()

    return kern(idx_flat, zeros_tile)


def kernel(atomic_numbers):
    idx_flat = atomic_numbers.astype(jnp.int32).reshape(-1)
    n_rows, row_w = atomic_numbers.shape
    flat = _onehot_sc(idx_flat)
    return flat.reshape(n_rows, row_w, NUM_CLASSES)
